# Initial kernel scaffold; baseline (speedup 1.0000x reference)
import jax
import jax.numpy as jnp
from jax.experimental import pallas as pl


def kernel(x, edge_index, W1, b1, W2, b2):
    raise NotImplementedError("write your pallas kernel here")


# speedup vs baseline: 11.3906x; 11.3906x over previous
import functools
import math

import jax
import jax.numpy as jnp
from jax import lax
from jax.experimental import pallas as pl
from jax.experimental.pallas import tpu as pltpu
from jax.experimental.pallas import tpu_sc as plsc

NC = 2
NS = 16
NW = NC * NS
CHUNK = 80


@functools.cache
def _seg_sum_kernel(n_nodes, d, n_edges):
    epw = n_edges // NW
    nch = epw // CHUNK
    rpt = n_nodes // NS
    mesh = plsc.VectorSubcoreMesh(core_axis_name="c", subcore_axis_name="s")

    @functools.partial(
        pl.kernel,
        out_type=jax.ShapeDtypeStruct((NC, n_nodes, d), jnp.float32),
        mesh=mesh,
        compiler_params=pltpu.CompilerParams(use_tc_tiling_on_sc=False),
        scratch_types=[
            pltpu.VMEM((nch, CHUNK), jnp.int32),
            pltpu.VMEM((nch, CHUNK), jnp.int32),
            pltpu.VMEM((CHUNK, d), jnp.float32),
            pltpu.VMEM_SHARED((n_nodes, d), jnp.float32),
            pltpu.SemaphoreType.DMA,
        ],
    )
    def seg(table_hbm, src_hbm, dst_hbm, zeros_hbm, out_hbm,
            src_v, dst_v, rows_v, acc_sh, sem):
        cid = lax.axis_index("c")
        sid = lax.axis_index("s")
        wid = sid * NC + cid

        pltpu.sync_copy(src_hbm.at[wid], src_v)
        pltpu.sync_copy(dst_hbm.at[wid], dst_v)

        row0 = sid * rpt
        pltpu.sync_copy(zeros_hbm.at[pl.ds(row0, rpt)],
                        acc_sh.at[pl.ds(row0, rpt)])
        plsc.subcore_barrier()

        def body(j, carry):
            pltpu.async_copy(table_hbm.at[src_v.at[j]], rows_v, sem).wait()
            pltpu.sync_copy(rows_v, acc_sh.at[dst_v.at[j]], add=True)
            return carry

        lax.fori_loop(0, nch, body, 0)
        plsc.subcore_barrier()

        pltpu.sync_copy(acc_sh.at[pl.ds(row0, rpt)],
                        out_hbm.at[cid, pl.ds(row0, rpt)])

    return seg


def _mm(x, w, blk):
    n, k = x.shape
    m = w.shape[1]

    def body(x_ref, w_ref, o_ref):
        o_ref[...] = jnp.dot(x_ref[...], w_ref[...],
                             preferred_element_type=jnp.float32)

    return pl.pallas_call(
        body,
        grid=(n // blk,),
        in_specs=[pl.BlockSpec((blk, k), lambda i: (i, 0)),
                  pl.BlockSpec((k, m), lambda i: (0, 0))],
        out_specs=pl.BlockSpec((blk, m), lambda i: (i, 0)),
        out_shape=jax.ShapeDtypeStruct((n, m), jnp.float32),
    )(x, w)


def _relu_combine(p, b, blk):
    _, n, d = p.shape

    def body(p_ref, b_ref, o_ref):
        o_ref[...] = jnp.maximum(p_ref[0] + p_ref[1] + b_ref[...], 0.0)

    return pl.pallas_call(
        body,
        grid=(n // blk,),
        in_specs=[pl.BlockSpec((2, blk, d), lambda i: (0, i, 0)),
                  pl.BlockSpec((1, d), lambda i: (0, 0))],
        out_specs=pl.BlockSpec((blk, d), lambda i: (i, 0)),
        out_shape=jax.ShapeDtypeStruct((n, d), jnp.float32),
    )(p, b)


def _combine_mm_bias(p, w, b, blk):
    _, n, k = p.shape
    m = w.shape[1]

    def body(p_ref, w_ref, b_ref, o_ref):
        s = p_ref[0] + p_ref[1]
        o_ref[...] = jnp.dot(s, w_ref[...],
                             preferred_element_type=jnp.float32) + b_ref[...]

    return pl.pallas_call(
        body,
        grid=(n // blk,),
        in_specs=[pl.BlockSpec((2, blk, k), lambda i: (0, i, 0)),
                  pl.BlockSpec((k, m), lambda i: (0, 0)),
                  pl.BlockSpec((1, m), lambda i: (0, 0))],
        out_specs=pl.BlockSpec((blk, m), lambda i: (i, 0)),
        out_shape=jax.ShapeDtypeStruct((n, m), jnp.float32),
    )(p, w, b)


def kernel(x, edge_index, W1, b1, W2, b2):
    n_nodes, _ = x.shape
    n_edges = edge_index.shape[1]
    d_hid = W1.shape[1]

    blk = 2048
    align = blk * (8 * NS) // math.gcd(blk, 8 * NS)
    n_pad = (n_nodes + align - 1) // align * align

    epw = n_edges // NW
    nch = epw // CHUNK
    src = edge_index[0].astype(jnp.int32).reshape(NW, nch, CHUNK)
    dst = edge_index[1].astype(jnp.int32).reshape(NW, nch, CHUNK)
    zeros = jnp.zeros((n_pad, d_hid), jnp.float32)

    seg = _seg_sum_kernel(n_pad, d_hid, n_edges)

    x_pad = jnp.pad(x, ((0, n_pad - n_nodes), (0, 0)))
    h1 = _mm(x_pad, W1, blk=blk)
    p1 = seg(h1, src, dst, zeros)
    h = _relu_combine(p1, b1.reshape(1, -1), blk=blk)
    p2 = seg(h, src, dst, zeros)
    out = _combine_mm_bias(p2, W2, b2.reshape(1, -1), blk=blk)
    return out[:n_nodes]


# speedup vs baseline: 23.0962x; 2.0277x over previous
import functools
import math

import jax
import jax.numpy as jnp
from jax import lax
from jax.experimental import pallas as pl
from jax.experimental.pallas import tpu as pltpu
from jax.experimental.pallas import tpu_sc as plsc

NC = 2
NS = 16
NW = NC * NS
CHUNK = 100
NBUF = 5


@functools.cache
def _seg_sum_kernel(n_nodes, d, n_edges):
    epw = n_edges // NW
    nch = epw // CHUNK
    rpt = n_nodes // NS
    mesh = plsc.VectorSubcoreMesh(core_axis_name="c", subcore_axis_name="s")

    assert nch % NBUF == 0 and nch // NBUF >= 2

    @functools.partial(
        pl.kernel,
        out_type=jax.ShapeDtypeStruct((NC, n_nodes, d), jnp.float32),
        mesh=mesh,
        compiler_params=pltpu.CompilerParams(use_tc_tiling_on_sc=False),
        scratch_types=[
            pltpu.VMEM((nch, CHUNK), jnp.int32),
            pltpu.VMEM((nch, CHUNK), jnp.int32),
            pltpu.VMEM((NBUF, CHUNK, d), jnp.float32),
            pltpu.VMEM_SHARED((n_nodes, d), jnp.float32),
        ] + [pltpu.SemaphoreType.DMA] * NBUF,
    )
    def seg(table_hbm, src_hbm, dst_hbm, zeros_hbm, out_hbm,
            src_v, dst_v, rows_v, acc_sh, *sems):
        cid = lax.axis_index("c")
        sid = lax.axis_index("s")
        wid = sid * NC + cid

        pltpu.sync_copy(src_hbm.at[wid], src_v)
        pltpu.sync_copy(dst_hbm.at[wid], dst_v)

        row0 = sid * rpt
        pltpu.sync_copy(zeros_hbm.at[pl.ds(row0, rpt)],
                        acc_sh.at[pl.ds(row0, rpt)])
        plsc.subcore_barrier()

        def gather(j, b):
            pltpu.async_copy(table_hbm.at[src_v.at[j]], rows_v.at[b], sems[b])

        def gwait(b):
            pltpu.make_async_copy(table_hbm.at[src_v.at[0]],
                                  rows_v.at[b], sems[b]).wait()

        def scatter(j, b):
            pltpu.sync_copy(rows_v.at[b], acc_sh.at[dst_v.at[j]], add=True)

        for b in range(NBUF):
            gather(b, b)

        def body(g, carry):
            j0 = g * NBUF
            for b in range(NBUF):
                gwait(b)
                scatter(j0 + b, b)
                gather(j0 + b + NBUF, b)
            return carry

        lax.fori_loop(0, nch // NBUF - 1, body, 0)
        j0 = nch - NBUF
        for b in range(NBUF):
            gwait(b)
            scatter(j0 + b, b)
        plsc.subcore_barrier()

        pltpu.sync_copy(acc_sh.at[pl.ds(row0, rpt)],
                        out_hbm.at[cid, pl.ds(row0, rpt)])

    return seg


def _mm(x, w, blk):
    n, k = x.shape
    m = w.shape[1]

    def body(x_ref, w_ref, o_ref):
        o_ref[...] = jnp.dot(x_ref[...], w_ref[...],
                             preferred_element_type=jnp.float32)

    return pl.pallas_call(
        body,
        grid=(n // blk,),
        in_specs=[pl.BlockSpec((blk, k), lambda i: (i, 0)),
                  pl.BlockSpec((k, m), lambda i: (0, 0))],
        out_specs=pl.BlockSpec((blk, m), lambda i: (i, 0)),
        out_shape=jax.ShapeDtypeStruct((n, m), jnp.float32),
    )(x, w)


def _relu_combine(p, b, blk):
    _, n, d = p.shape

    def body(p_ref, b_ref, o_ref):
        o_ref[...] = jnp.maximum(p_ref[0] + p_ref[1] + b_ref[...], 0.0)

    return pl.pallas_call(
        body,
        grid=(n // blk,),
        in_specs=[pl.BlockSpec((2, blk, d), lambda i: (0, i, 0)),
                  pl.BlockSpec((1, d), lambda i: (0, 0))],
        out_specs=pl.BlockSpec((blk, d), lambda i: (i, 0)),
        out_shape=jax.ShapeDtypeStruct((n, d), jnp.float32),
    )(p, b)


def _combine_mm_bias(p, w, b, blk):
    _, n, k = p.shape
    m = w.shape[1]

    def body(p_ref, w_ref, b_ref, o_ref):
        s = p_ref[0] + p_ref[1]
        o_ref[...] = jnp.dot(s, w_ref[...],
                             preferred_element_type=jnp.float32) + b_ref[...]

    return pl.pallas_call(
        body,
        grid=(n // blk,),
        in_specs=[pl.BlockSpec((2, blk, k), lambda i: (0, i, 0)),
                  pl.BlockSpec((k, m), lambda i: (0, 0)),
                  pl.BlockSpec((1, m), lambda i: (0, 0))],
        out_specs=pl.BlockSpec((blk, m), lambda i: (i, 0)),
        out_shape=jax.ShapeDtypeStruct((n, m), jnp.float32),
    )(p, w, b)


def kernel(x, edge_index, W1, b1, W2, b2):
    n_nodes, _ = x.shape
    n_edges = edge_index.shape[1]
    d_hid = W1.shape[1]

    blk = 2048
    align = blk * (8 * NS) // math.gcd(blk, 8 * NS)
    n_pad = (n_nodes + align - 1) // align * align

    epw = n_edges // NW
    nch = epw // CHUNK
    src = edge_index[0].astype(jnp.int32).reshape(NW, nch, CHUNK)
    dst = edge_index[1].astype(jnp.int32).reshape(NW, nch, CHUNK)
    zeros = jnp.zeros((n_pad, d_hid), jnp.float32)

    seg = _seg_sum_kernel(n_pad, d_hid, n_edges)

    x_pad = jnp.pad(x, ((0, n_pad - n_nodes), (0, 0)))
    h1 = _mm(x_pad, W1, blk=blk)
    p1 = seg(h1, src, dst, zeros)
    h = _relu_combine(p1, b1.reshape(1, -1), blk=blk)
    p2 = seg(h, src, dst, zeros)
    out = _combine_mm_bias(p2, W2, b2.reshape(1, -1), blk=blk)
    return out[:n_nodes]


# speedup vs baseline: 23.7434x; 1.0280x over previous
import functools
import math

import jax
import jax.numpy as jnp
from jax import lax
from jax.experimental import pallas as pl
from jax.experimental.pallas import tpu as pltpu
from jax.experimental.pallas import tpu_sc as plsc

NC = 2
NS = 16
NW = NC * NS
CHUNK = 100
NBUF = 10


@functools.cache
def _seg_sum_kernel(n_nodes, d, n_edges):
    epw = n_edges // NW
    nch = epw // CHUNK
    rpt = n_nodes // NS
    mesh = plsc.VectorSubcoreMesh(core_axis_name="c", subcore_axis_name="s")

    assert nch % NBUF == 0 and nch // NBUF >= 2

    @functools.partial(
        pl.kernel,
        out_type=jax.ShapeDtypeStruct((NC, n_nodes, d), jnp.float32),
        mesh=mesh,
        compiler_params=pltpu.CompilerParams(use_tc_tiling_on_sc=False),
        scratch_types=[
            pltpu.VMEM((nch, CHUNK), jnp.int32),
            pltpu.VMEM((nch, CHUNK), jnp.int32),
            pltpu.VMEM((NBUF, CHUNK, d), jnp.float32),
            pltpu.VMEM_SHARED((n_nodes, d), jnp.float32),
        ] + [pltpu.SemaphoreType.DMA] * (2 * NBUF),
    )
    def seg(table_hbm, src_hbm, dst_hbm, zeros_hbm, out_hbm,
            src_v, dst_v, rows_v, acc_sh, *sems):
        cid = lax.axis_index("c")
        sid = lax.axis_index("s")
        wid = sid * NC + cid

        pltpu.sync_copy(src_hbm.at[wid], src_v)
        pltpu.sync_copy(dst_hbm.at[wid], dst_v)

        row0 = sid * rpt
        pltpu.sync_copy(zeros_hbm.at[pl.ds(row0, rpt)],
                        acc_sh.at[pl.ds(row0, rpt)])
        plsc.subcore_barrier()

        gsems = sems[:NBUF]
        ssems = sems[NBUF:]

        def gather(j, b):
            pltpu.async_copy(table_hbm.at[src_v.at[j]], rows_v.at[b], gsems[b])

        def gwait(b):
            pltpu.make_async_copy(table_hbm.at[src_v.at[0]],
                                  rows_v.at[b], gsems[b]).wait()

        def scatter(j, b):
            pltpu.async_copy(rows_v.at[b], acc_sh.at[dst_v.at[j]], ssems[b],
                             add=True)

        def swait(b):
            pltpu.make_async_copy(rows_v.at[b], acc_sh.at[dst_v.at[0]],
                                  ssems[b]).wait()

        for b in range(NBUF):
            gather(b, b)

        def body(g, carry):
            j0 = g * NBUF
            for b in range(NBUF):
                gwait(b)
                scatter(j0 + b, b)
            for b in range(NBUF):
                swait(b)
                gather(j0 + b + NBUF, b)
            return carry

        lax.fori_loop(0, nch // NBUF - 1, body, 0)
        j0 = nch - NBUF
        for b in range(NBUF):
            gwait(b)
            scatter(j0 + b, b)
        for b in range(NBUF):
            swait(b)
        plsc.subcore_barrier()

        pltpu.sync_copy(acc_sh.at[pl.ds(row0, rpt)],
                        out_hbm.at[cid, pl.ds(row0, rpt)])

    return seg


def _mm(x, w, blk):
    n, k = x.shape
    m = w.shape[1]

    def body(x_ref, w_ref, o_ref):
        o_ref[...] = jnp.dot(x_ref[...], w_ref[...],
                             preferred_element_type=jnp.float32)

    return pl.pallas_call(
        body,
        grid=(n // blk,),
        in_specs=[pl.BlockSpec((blk, k), lambda i: (i, 0)),
                  pl.BlockSpec((k, m), lambda i: (0, 0))],
        out_specs=pl.BlockSpec((blk, m), lambda i: (i, 0)),
        out_shape=jax.ShapeDtypeStruct((n, m), jnp.float32),
    )(x, w)


def _relu_combine(p, b, blk):
    _, n, d = p.shape

    def body(p_ref, b_ref, o_ref):
        o_ref[...] = jnp.maximum(p_ref[0] + p_ref[1] + b_ref[...], 0.0)

    return pl.pallas_call(
        body,
        grid=(n // blk,),
        in_specs=[pl.BlockSpec((2, blk, d), lambda i: (0, i, 0)),
                  pl.BlockSpec((1, d), lambda i: (0, 0))],
        out_specs=pl.BlockSpec((blk, d), lambda i: (i, 0)),
        out_shape=jax.ShapeDtypeStruct((n, d), jnp.float32),
    )(p, b)


def _combine_mm_bias(p, w, b, blk):
    _, n, k = p.shape
    m = w.shape[1]

    def body(p_ref, w_ref, b_ref, o_ref):
        s = p_ref[0] + p_ref[1]
        o_ref[...] = jnp.dot(s, w_ref[...],
                             preferred_element_type=jnp.float32) + b_ref[...]

    return pl.pallas_call(
        body,
        grid=(n // blk,),
        in_specs=[pl.BlockSpec((2, blk, k), lambda i: (0, i, 0)),
                  pl.BlockSpec((k, m), lambda i: (0, 0)),
                  pl.BlockSpec((1, m), lambda i: (0, 0))],
        out_specs=pl.BlockSpec((blk, m), lambda i: (i, 0)),
        out_shape=jax.ShapeDtypeStruct((n, m), jnp.float32),
    )(p, w, b)


def kernel(x, edge_index, W1, b1, W2, b2):
    n_nodes, _ = x.shape
    n_edges = edge_index.shape[1]
    d_hid = W1.shape[1]

    blk = 2048
    align = blk * (8 * NS) // math.gcd(blk, 8 * NS)
    n_pad = (n_nodes + align - 1) // align * align

    epw = n_edges // NW
    nch = epw // CHUNK
    src = edge_index[0].astype(jnp.int32).reshape(NW, nch, CHUNK)
    dst = edge_index[1].astype(jnp.int32).reshape(NW, nch, CHUNK)
    zeros = jnp.zeros((n_pad, d_hid), jnp.float32)

    seg = _seg_sum_kernel(n_pad, d_hid, n_edges)

    x_pad = jnp.pad(x, ((0, n_pad - n_nodes), (0, 0)))
    h1 = _mm(x_pad, W1, blk=blk)
    p1 = seg(h1, src, dst, zeros)
    h = _relu_combine(p1, b1.reshape(1, -1), blk=blk)
    p2 = seg(h, src, dst, zeros)
    out = _combine_mm_bias(p2, W2, b2.reshape(1, -1), blk=blk)
    return out[:n_nodes]


# speedup vs baseline: 27.3870x; 1.1535x over previous
import functools

import jax
import jax.numpy as jnp
from jax import lax
from jax.experimental import pallas as pl
from jax.experimental.pallas import tpu as pltpu
from jax.experimental.pallas import tpu_sc as plsc

NC = 2
NS = 16
NW = NC * NS
CHUNK = 100
NBUF = 10


@functools.cache
def _seg_sum_kernel(n_nodes, d, n_edges):
    epw = n_edges // NW
    nch = epw // CHUNK
    rpt = n_nodes // NS
    mesh = plsc.VectorSubcoreMesh(core_axis_name="c", subcore_axis_name="s")

    assert nch % NBUF == 0 and nch // NBUF >= 2

    @functools.partial(
        pl.kernel,
        out_type=jax.ShapeDtypeStruct((NC, n_nodes, d), jnp.float32),
        mesh=mesh,
        compiler_params=pltpu.CompilerParams(use_tc_tiling_on_sc=False),
        scratch_types=[
            pltpu.VMEM((nch, CHUNK), jnp.int32),
            pltpu.VMEM((nch, CHUNK), jnp.int32),
            pltpu.VMEM((NBUF, CHUNK, d), jnp.float32),
            pltpu.VMEM_SHARED((n_nodes, d), jnp.float32),
        ] + [pltpu.SemaphoreType.DMA] * (2 * NBUF),
    )
    def seg(table_hbm, src_hbm, dst_hbm, zeros_hbm, out_hbm,
            src_v, dst_v, rows_v, acc_sh, *sems):
        cid = lax.axis_index("c")
        sid = lax.axis_index("s")
        wid = sid * NC + cid

        pltpu.sync_copy(src_hbm.at[wid], src_v)
        pltpu.sync_copy(dst_hbm.at[wid], dst_v)

        row0 = sid * rpt
        pltpu.sync_copy(zeros_hbm.at[pl.ds(row0, rpt)],
                        acc_sh.at[pl.ds(row0, rpt)])
        plsc.subcore_barrier()

        gsems = sems[:NBUF]
        ssems = sems[NBUF:]

        def gather(j, b):
            pltpu.async_copy(table_hbm.at[src_v.at[j]], rows_v.at[b], gsems[b])

        def gwait(b):
            pltpu.make_async_copy(table_hbm.at[src_v.at[0]],
                                  rows_v.at[b], gsems[b]).wait()

        def scatter(j, b):
            pltpu.async_copy(rows_v.at[b], acc_sh.at[dst_v.at[j]], ssems[b],
                             add=True)

        def swait(b):
            pltpu.make_async_copy(rows_v.at[b], acc_sh.at[dst_v.at[0]],
                                  ssems[b]).wait()

        for b in range(NBUF):
            gather(b, b)

        def body(g, carry):
            j0 = g * NBUF
            for b in range(NBUF):
                gwait(b)
                scatter(j0 + b, b)
            for b in range(NBUF):
                swait(b)
                gather(j0 + b + NBUF, b)
            return carry

        lax.fori_loop(0, nch // NBUF - 1, body, 0)
        j0 = nch - NBUF
        for b in range(NBUF):
            gwait(b)
            scatter(j0 + b, b)
        for b in range(NBUF):
            swait(b)
        plsc.subcore_barrier()

        pltpu.sync_copy(acc_sh.at[pl.ds(row0, rpt)],
                        out_hbm.at[cid, pl.ds(row0, rpt)])

    return seg


@functools.cache
def _relu_combine_sc_kernel(n_nodes, d):
    assert d == 16 and n_nodes % (8 * NW) == 0
    rpt = n_nodes // NW
    mesh = plsc.VectorSubcoreMesh(core_axis_name="c", subcore_axis_name="s")

    @functools.partial(
        pl.kernel,
        out_type=jax.ShapeDtypeStruct((n_nodes, d), jnp.float32),
        mesh=mesh,
        compiler_params=pltpu.CompilerParams(use_tc_tiling_on_sc=False),
        scratch_types=[
            pltpu.VMEM((rpt, d), jnp.float32),
            pltpu.VMEM((rpt, d), jnp.float32),
            pltpu.VMEM((d,), jnp.float32),
        ],
    )
    def relu_k(p_hbm, b_hbm, out_hbm, a_v, c_v, bias_v):
        cid = lax.axis_index("c")
        sid = lax.axis_index("s")
        wid = sid * NC + cid
        r0 = wid * rpt
        pltpu.sync_copy(p_hbm.at[0, pl.ds(r0, rpt)], a_v)
        pltpu.sync_copy(p_hbm.at[1, pl.ds(r0, rpt)], c_v)
        pltpu.sync_copy(b_hbm, bias_v)
        bias = bias_v[...]

        def body(i, carry):
            a_v[i] = jnp.maximum(a_v[i] + c_v[i] + bias, 0.0)
            return carry

        lax.fori_loop(0, rpt, body, 0)
        pltpu.sync_copy(a_v, out_hbm.at[pl.ds(r0, rpt)])

    return relu_k


def _mm(x, w, blk):
    n, k = x.shape
    m = w.shape[1]

    def body(x_ref, w_ref, o_ref):
        o_ref[...] = jnp.dot(x_ref[...], w_ref[...],
                             preferred_element_type=jnp.float32)

    return pl.pallas_call(
        body,
        grid=(n // blk,),
        in_specs=[pl.BlockSpec((blk, k), lambda i: (i, 0)),
                  pl.BlockSpec((k, m), lambda i: (0, 0))],
        out_specs=pl.BlockSpec((blk, m), lambda i: (i, 0)),
        out_shape=jax.ShapeDtypeStruct((n, m), jnp.float32),
    )(x, w)


def _combine_mm_bias(p, w, b, blk, n):
    k = p.shape[2]
    m = w.shape[1]

    def body(p_ref, w_ref, b_ref, o_ref):
        s = p_ref[0] + p_ref[1]
        o_ref[...] = jnp.dot(s, w_ref[...],
                             preferred_element_type=jnp.float32) + b_ref[...]

    return pl.pallas_call(
        body,
        grid=(n // blk,),
        in_specs=[pl.BlockSpec((2, blk, k), lambda i: (0, i, 0)),
                  pl.BlockSpec((k, m), lambda i: (0, 0)),
                  pl.BlockSpec((1, m), lambda i: (0, 0))],
        out_specs=pl.BlockSpec((blk, m), lambda i: (i, 0)),
        out_shape=jax.ShapeDtypeStruct((n, m), jnp.float32),
    )(p, w, b)


def kernel(x, edge_index, W1, b1, W2, b2):
    n_nodes, _ = x.shape
    n_edges = edge_index.shape[1]
    d_hid = W1.shape[1]

    blk = 2000
    assert n_nodes % blk == 0
    align = 8 * NW
    n_pad = (n_nodes + align - 1) // align * align

    epw = n_edges // NW
    nch = epw // CHUNK
    src = edge_index[0].astype(jnp.int32).reshape(NW, nch, CHUNK)
    dst = edge_index[1].astype(jnp.int32).reshape(NW, nch, CHUNK)
    zeros = jnp.zeros((n_pad, d_hid), jnp.float32)

    seg = _seg_sum_kernel(n_pad, d_hid, n_edges)
    relu_k = _relu_combine_sc_kernel(n_pad, d_hid)

    h1 = _mm(x, W1, blk=blk)
    p1 = seg(h1, src, dst, zeros)
    h = relu_k(p1, b1)
    p2 = seg(h, src, dst, zeros)
    return _combine_mm_bias(p2, W2, b2.reshape(1, -1), blk=blk, n=n_nodes)


# speedup vs baseline: 29.0753x; 1.0616x over previous
import functools

import jax
import jax.numpy as jnp
from jax import lax
from jax.experimental import pallas as pl
from jax.experimental.pallas import tpu as pltpu
from jax.experimental.pallas import tpu_sc as plsc

NC = 2
NS = 16
NW = NC * NS
CHUNK = 100
NBUF = 10


@functools.cache
def _seg_sum_kernel(n_nodes, d, n_edges):
    epw = n_edges // NW
    nch = epw // CHUNK
    rpt = n_nodes // NS
    mesh = plsc.VectorSubcoreMesh(core_axis_name="c", subcore_axis_name="s")

    assert nch % NBUF == 0 and nch // NBUF >= 2

    @functools.partial(
        pl.kernel,
        out_type=jax.ShapeDtypeStruct((NC, n_nodes, d), jnp.float32),
        mesh=mesh,
        compiler_params=pltpu.CompilerParams(use_tc_tiling_on_sc=False),
        scratch_types=[
            pltpu.VMEM((nch, CHUNK), jnp.int32),
            pltpu.VMEM((nch, CHUNK), jnp.int32),
            pltpu.VMEM((NBUF, CHUNK, d), jnp.float32),
            pltpu.VMEM_SHARED((n_nodes, d), jnp.float32),
        ] + [pltpu.SemaphoreType.DMA] * (2 * NBUF),
    )
    def seg(table_hbm, edge_hbm, zeros_hbm, out_hbm,
            src_v, dst_v, rows_v, acc_sh, *sems):
        cid = lax.axis_index("c")
        sid = lax.axis_index("s")
        wid = sid * NC + cid

        pltpu.sync_copy(edge_hbm.at[0, wid], src_v)
        pltpu.sync_copy(edge_hbm.at[1, wid], dst_v)

        row0 = sid * rpt
        pltpu.sync_copy(zeros_hbm.at[pl.ds(row0, rpt)],
                        acc_sh.at[pl.ds(row0, rpt)])
        plsc.subcore_barrier()

        gsems = sems[:NBUF]
        ssems = sems[NBUF:]

        def gather(j, b):
            pltpu.async_copy(table_hbm.at[src_v.at[j]], rows_v.at[b], gsems[b])

        def gwait(b):
            pltpu.make_async_copy(table_hbm.at[src_v.at[0]],
                                  rows_v.at[b], gsems[b]).wait()

        def scatter(j, b):
            pltpu.async_copy(rows_v.at[b], acc_sh.at[dst_v.at[j]], ssems[b],
                             add=True)

        def swait(b):
            pltpu.make_async_copy(rows_v.at[b], acc_sh.at[dst_v.at[0]],
                                  ssems[b]).wait()

        for b in range(NBUF):
            gather(b, b)

        def body(g, carry):
            j0 = g * NBUF
            for b in range(NBUF):
                gwait(b)
                scatter(j0 + b, b)
            for b in range(NBUF):
                swait(b)
                gather(j0 + b + NBUF, b)
            return carry

        lax.fori_loop(0, nch // NBUF - 1, body, 0)
        j0 = nch - NBUF
        for b in range(NBUF):
            gwait(b)
            scatter(j0 + b, b)
        for b in range(NBUF):
            swait(b)
        plsc.subcore_barrier()

        pltpu.sync_copy(acc_sh.at[pl.ds(row0, rpt)],
                        out_hbm.at[cid, pl.ds(row0, rpt)])

    return seg


@functools.cache
def _relu_combine_sc_kernel(n_nodes, d):
    assert d == 16 and n_nodes % (8 * NW) == 0
    rpt = n_nodes // NW
    mesh = plsc.VectorSubcoreMesh(core_axis_name="c", subcore_axis_name="s")

    @functools.partial(
        pl.kernel,
        out_type=jax.ShapeDtypeStruct((n_nodes, d), jnp.float32),
        mesh=mesh,
        compiler_params=pltpu.CompilerParams(use_tc_tiling_on_sc=False),
        scratch_types=[
            pltpu.VMEM((rpt, d), jnp.float32),
            pltpu.VMEM((rpt, d), jnp.float32),
            pltpu.VMEM((d,), jnp.float32),
        ],
    )
    def relu_k(p_hbm, b_hbm, out_hbm, a_v, c_v, bias_v):
        cid = lax.axis_index("c")
        sid = lax.axis_index("s")
        wid = sid * NC + cid
        r0 = wid * rpt
        pltpu.sync_copy(p_hbm.at[0, pl.ds(r0, rpt)], a_v)
        pltpu.sync_copy(p_hbm.at[1, pl.ds(r0, rpt)], c_v)
        pltpu.sync_copy(b_hbm, bias_v)
        bias = bias_v[...]

        def body(i, carry):
            a_v[i] = jnp.maximum(a_v[i] + c_v[i] + bias, 0.0)
            return carry

        lax.fori_loop(0, rpt, body, 0)
        pltpu.sync_copy(a_v, out_hbm.at[pl.ds(r0, rpt)])

    return relu_k


def _mm(x, w, blk):
    n, k = x.shape
    m = w.shape[1]

    def body(x_ref, w_ref, o_ref):
        o_ref[...] = jnp.dot(x_ref[...], w_ref[...],
                             preferred_element_type=jnp.float32)

    return pl.pallas_call(
        body,
        grid=(n // blk,),
        in_specs=[pl.BlockSpec((blk, k), lambda i: (i, 0)),
                  pl.BlockSpec((k, m), lambda i: (0, 0))],
        out_specs=pl.BlockSpec((blk, m), lambda i: (i, 0)),
        out_shape=jax.ShapeDtypeStruct((n, m), jnp.float32),
    )(x, w)


def _combine_mm_bias(p, w, b, blk, n):
    k = p.shape[2]
    m = w.shape[1]

    def body(p_ref, w_ref, b_ref, o_ref):
        s = p_ref[0] + p_ref[1]
        o_ref[...] = jnp.dot(s, w_ref[...],
                             preferred_element_type=jnp.float32) + b_ref[...]

    return pl.pallas_call(
        body,
        grid=(n // blk,),
        in_specs=[pl.BlockSpec((2, blk, k), lambda i: (0, i, 0)),
                  pl.BlockSpec((k, m), lambda i: (0, 0)),
                  pl.BlockSpec((1, m), lambda i: (0, 0))],
        out_specs=pl.BlockSpec((blk, m), lambda i: (i, 0)),
        out_shape=jax.ShapeDtypeStruct((n, m), jnp.float32),
    )(p, w, b)


def kernel(x, edge_index, W1, b1, W2, b2):
    n_nodes, _ = x.shape
    n_edges = edge_index.shape[1]
    d_hid = W1.shape[1]

    blk = 2000
    assert n_nodes % blk == 0
    align = 8 * NW
    n_pad = (n_nodes + align - 1) // align * align

    epw = n_edges // NW
    nch = epw // CHUNK
    edges = edge_index.astype(jnp.int32).reshape(2, NW, nch, CHUNK)
    zeros = jnp.zeros((n_pad, d_hid), jnp.float32)

    seg = _seg_sum_kernel(n_pad, d_hid, n_edges)
    relu_k = _relu_combine_sc_kernel(n_pad, d_hid)

    h1 = _mm(x, W1, blk=blk)
    p1 = seg(h1, edges, zeros)
    h = relu_k(p1, b1)
    p2 = seg(h, edges, zeros)
    return _combine_mm_bias(p2, W2, b2.reshape(1, -1), blk=blk, n=n_nodes)


# speedup vs baseline: 30.3436x; 1.0436x over previous
import functools

import jax
import jax.numpy as jnp
from jax import lax
from jax.experimental import pallas as pl
from jax.experimental.pallas import tpu as pltpu
from jax.experimental.pallas import tpu_sc as plsc

NC = 2
NS = 16
NW = NC * NS
CHUNK = 100
NBUF = 10


@functools.cache
def _seg_sum_kernel(n_nodes, d, n_edges):
    epw = n_edges // NW
    nch = epw // CHUNK
    rpt = n_nodes // NS
    mesh = plsc.VectorSubcoreMesh(core_axis_name="c", subcore_axis_name="s")

    assert nch % NBUF == 0 and nch // NBUF >= 2

    @functools.partial(
        pl.kernel,
        out_type=jax.ShapeDtypeStruct((NC, n_nodes, d), jnp.float32),
        mesh=mesh,
        compiler_params=pltpu.CompilerParams(use_tc_tiling_on_sc=False),
        scratch_types=[
            pltpu.VMEM((nch, CHUNK), jnp.int32),
            pltpu.VMEM((nch, CHUNK), jnp.int32),
            pltpu.VMEM((NBUF, CHUNK, d), jnp.float32),
            pltpu.VMEM_SHARED((n_nodes, d), jnp.float32),
        ] + [pltpu.SemaphoreType.DMA] * (2 * NBUF),
    )
    def seg(table_hbm, edge_hbm, zeros_hbm, out_hbm,
            src_v, dst_v, rows_v, acc_sh, *sems):
        cid = lax.axis_index("c")
        sid = lax.axis_index("s")
        wid = sid * NC + cid

        pltpu.sync_copy(edge_hbm.at[0, wid], src_v)
        pltpu.sync_copy(edge_hbm.at[1, wid], dst_v)

        row0 = sid * rpt
        pltpu.sync_copy(zeros_hbm.at[pl.ds(row0, rpt)],
                        acc_sh.at[pl.ds(row0, rpt)])
        plsc.subcore_barrier()

        gsems = sems[:NBUF]
        ssems = sems[NBUF:]

        def gather(j, b):
            pltpu.async_copy(table_hbm.at[src_v.at[j]], rows_v.at[b], gsems[b])

        def gwait(b):
            pltpu.make_async_copy(table_hbm.at[src_v.at[0]],
                                  rows_v.at[b], gsems[b]).wait()

        def scatter(j, b):
            pltpu.async_copy(rows_v.at[b], acc_sh.at[dst_v.at[j]], ssems[b],
                             add=True)

        def swait(b):
            pltpu.make_async_copy(rows_v.at[b], acc_sh.at[dst_v.at[0]],
                                  ssems[b]).wait()

        for b in range(NBUF):
            gather(b, b)

        def body(g, carry):
            j0 = g * NBUF
            for b in range(NBUF):
                gwait(b)
                scatter(j0 + b, b)
            for b in range(NBUF):
                swait(b)
                gather(j0 + b + NBUF, b)
            return carry

        lax.fori_loop(0, nch // NBUF - 1, body, 0)
        j0 = nch - NBUF
        for b in range(NBUF):
            gwait(b)
            scatter(j0 + b, b)
        for b in range(NBUF):
            swait(b)
        plsc.subcore_barrier()

        pltpu.sync_copy(acc_sh.at[pl.ds(row0, rpt)],
                        out_hbm.at[cid, pl.ds(row0, rpt)])

    return seg


@functools.cache
def _relu_combine_sc_kernel(n_nodes, d):
    assert d == 16 and n_nodes % (8 * NW) == 0
    rpt = n_nodes // NW
    mesh = plsc.VectorSubcoreMesh(core_axis_name="c", subcore_axis_name="s")

    @functools.partial(
        pl.kernel,
        out_type=jax.ShapeDtypeStruct((n_nodes, d), jnp.float32),
        mesh=mesh,
        compiler_params=pltpu.CompilerParams(use_tc_tiling_on_sc=False),
        scratch_types=[
            pltpu.VMEM((rpt, d), jnp.float32),
            pltpu.VMEM((rpt, d), jnp.float32),
            pltpu.VMEM((d,), jnp.float32),
        ],
    )
    def relu_k(p_hbm, b_hbm, out_hbm, a_v, c_v, bias_v):
        cid = lax.axis_index("c")
        sid = lax.axis_index("s")
        wid = sid * NC + cid
        r0 = wid * rpt
        pltpu.sync_copy(p_hbm.at[0, pl.ds(r0, rpt)], a_v)
        pltpu.sync_copy(p_hbm.at[1, pl.ds(r0, rpt)], c_v)
        pltpu.sync_copy(b_hbm, bias_v)
        bias = bias_v[...]

        UNROLL = 8

        def body(i, carry):
            for u in range(UNROLL):
                r = i * UNROLL + u
                a_v[r] = jnp.maximum(a_v[r] + c_v[r] + bias, 0.0)
            return carry

        lax.fori_loop(0, rpt // UNROLL, body, 0)
        pltpu.sync_copy(a_v, out_hbm.at[pl.ds(r0, rpt)])

    return relu_k


def _mm(x, w):
    n, k = x.shape
    m = w.shape[1]

    def body(x_ref, w_ref, o_ref):
        o_ref[...] = jnp.dot(x_ref[...], w_ref[...],
                             preferred_element_type=jnp.float32)

    return pl.pallas_call(
        body,
        out_shape=jax.ShapeDtypeStruct((n, m), jnp.float32),
    )(x, w)


def _combine_mm_bias(p, w, b, n):
    npad, k = p.shape[1], p.shape[2]
    m = w.shape[1]

    def body(p_ref, w_ref, b_ref, o_ref):
        s = p_ref[0] + p_ref[1]
        o_ref[...] = jnp.dot(s, w_ref[...],
                             preferred_element_type=jnp.float32) + b_ref[...]

    return pl.pallas_call(
        body,
        grid=(1,),
        in_specs=[pl.BlockSpec((2, n, k), lambda i: (0, 0, 0)),
                  pl.BlockSpec((k, m), lambda i: (0, 0)),
                  pl.BlockSpec((1, m), lambda i: (0, 0))],
        out_specs=pl.BlockSpec((n, m), lambda i: (0, 0)),
        out_shape=jax.ShapeDtypeStruct((n, m), jnp.float32),
    )(p, w, b)


def kernel(x, edge_index, W1, b1, W2, b2):
    n_nodes, _ = x.shape
    n_edges = edge_index.shape[1]
    d_hid = W1.shape[1]

    align = 8 * NW
    n_pad = (n_nodes + align - 1) // align * align

    epw = n_edges // NW
    nch = epw // CHUNK
    edges = edge_index.astype(jnp.int32).reshape(2, NW, nch, CHUNK)
    zeros = jnp.zeros((n_pad, d_hid), jnp.float32)

    seg = _seg_sum_kernel(n_pad, d_hid, n_edges)
    relu_k = _relu_combine_sc_kernel(n_pad, d_hid)

    h1 = _mm(x, W1)
    p1 = seg(h1, edges, zeros)
    h = relu_k(p1, b1)
    p2 = seg(h, edges, zeros)
    return _combine_mm_bias(p2, W2, b2.reshape(1, -1), n=n_nodes)


# speedup vs baseline: 31.1890x; 1.0279x over previous
import functools

import jax
import jax.numpy as jnp
from jax import lax
from jax.experimental import pallas as pl
from jax.experimental.pallas import tpu as pltpu
from jax.experimental.pallas import tpu_sc as plsc

NC = 2
NS = 16
NW = NC * NS
CHUNK = 100
NBUF = 10


@functools.cache
def _seg_sum_kernel(n_nodes, d, n_edges):
    epw = n_edges // NW
    nch = epw // CHUNK
    rpt = n_nodes // NS
    mesh = plsc.VectorSubcoreMesh(core_axis_name="c", subcore_axis_name="s")

    assert nch % NBUF == 0 and nch // NBUF >= 2

    @functools.partial(
        pl.kernel,
        out_type=jax.ShapeDtypeStruct((NC, n_nodes, d), jnp.float32),
        mesh=mesh,
        compiler_params=pltpu.CompilerParams(use_tc_tiling_on_sc=False),
        scratch_types=[
            pltpu.VMEM((nch, CHUNK), jnp.int32),
            pltpu.VMEM((nch, CHUNK), jnp.int32),
            pltpu.VMEM((NBUF, CHUNK, d), jnp.float32),
            pltpu.VMEM((rpt, d), jnp.float32),
            pltpu.VMEM_SHARED((n_nodes, d), jnp.float32),
        ] + [pltpu.SemaphoreType.DMA] * (2 * NBUF),
    )
    def seg(table_hbm, edge_hbm, out_hbm,
            src_v, dst_v, rows_v, zero_v, acc_sh, *sems):
        cid = lax.axis_index("c")
        sid = lax.axis_index("s")
        wid = sid * NC + cid

        pltpu.sync_copy(edge_hbm.at[0, wid], src_v)
        pltpu.sync_copy(edge_hbm.at[1, wid], dst_v)

        gsems = sems[:NBUF]
        ssems = sems[NBUF:]

        def gather(j, b):
            pltpu.async_copy(table_hbm.at[src_v.at[j]], rows_v.at[b], gsems[b])

        def gwait(b):
            pltpu.make_async_copy(table_hbm.at[src_v.at[0]],
                                  rows_v.at[b], gsems[b]).wait()

        def scatter(j, b):
            pltpu.async_copy(rows_v.at[b], acc_sh.at[dst_v.at[j]], ssems[b],
                             add=True)

        def swait(b):
            pltpu.make_async_copy(rows_v.at[b], acc_sh.at[dst_v.at[0]],
                                  ssems[b]).wait()

        for b in range(NBUF):
            gather(b, b)

        zrow = jnp.zeros((d,), jnp.float32)

        def zbody(i, carry):
            for u in range(8):
                zero_v[i * 8 + u] = zrow
            return carry

        lax.fori_loop(0, rpt // 8, zbody, 0)
        row0 = sid * rpt
        pltpu.sync_copy(zero_v, acc_sh.at[pl.ds(row0, rpt)])
        plsc.subcore_barrier()

        def body(g, carry):
            j0 = g * NBUF
            for b in range(NBUF):
                gwait(b)
                scatter(j0 + b, b)
            for b in range(NBUF):
                swait(b)
                gather(j0 + b + NBUF, b)
            return carry

        lax.fori_loop(0, nch // NBUF - 1, body, 0)
        j0 = nch - NBUF
        for b in range(NBUF):
            gwait(b)
            scatter(j0 + b, b)
        for b in range(NBUF):
            swait(b)
        plsc.subcore_barrier()

        pltpu.sync_copy(acc_sh.at[pl.ds(row0, rpt)],
                        out_hbm.at[cid, pl.ds(row0, rpt)])

    return seg


@functools.cache
def _relu_combine_sc_kernel(n_nodes, d):
    assert d == 16 and n_nodes % (8 * NW) == 0
    rpt = n_nodes // NW
    mesh = plsc.VectorSubcoreMesh(core_axis_name="c", subcore_axis_name="s")

    @functools.partial(
        pl.kernel,
        out_type=jax.ShapeDtypeStruct((n_nodes, d), jnp.float32),
        mesh=mesh,
        compiler_params=pltpu.CompilerParams(use_tc_tiling_on_sc=False),
        scratch_types=[
            pltpu.VMEM((rpt, d), jnp.float32),
            pltpu.VMEM((rpt, d), jnp.float32),
            pltpu.VMEM((d,), jnp.float32),
        ],
    )
    def relu_k(p_hbm, b_hbm, out_hbm, a_v, c_v, bias_v):
        cid = lax.axis_index("c")
        sid = lax.axis_index("s")
        wid = sid * NC + cid
        r0 = wid * rpt
        pltpu.sync_copy(p_hbm.at[0, pl.ds(r0, rpt)], a_v)
        pltpu.sync_copy(p_hbm.at[1, pl.ds(r0, rpt)], c_v)
        pltpu.sync_copy(b_hbm, bias_v)
        bias = bias_v[...]

        UNROLL = 8

        def body(i, carry):
            for u in range(UNROLL):
                r = i * UNROLL + u
                a_v[r] = jnp.maximum(a_v[r] + c_v[r] + bias, 0.0)
            return carry

        lax.fori_loop(0, rpt // UNROLL, body, 0)
        pltpu.sync_copy(a_v, out_hbm.at[pl.ds(r0, rpt)])

    return relu_k


def _mm(x, w):
    n, k = x.shape
    m = w.shape[1]

    def body(x_ref, w_ref, o_ref):
        o_ref[...] = jnp.dot(x_ref[...], w_ref[...],
                             preferred_element_type=jnp.float32)

    return pl.pallas_call(
        body,
        out_shape=jax.ShapeDtypeStruct((n, m), jnp.float32),
    )(x, w)


def _combine_mm_bias(p, w, b, n):
    npad, k = p.shape[1], p.shape[2]
    m = w.shape[1]

    def body(p_ref, w_ref, b_ref, o_ref):
        s = p_ref[0] + p_ref[1]
        o_ref[...] = jnp.dot(s, w_ref[...],
                             preferred_element_type=jnp.float32) + b_ref[...]

    return pl.pallas_call(
        body,
        grid=(1,),
        in_specs=[pl.BlockSpec((2, n, k), lambda i: (0, 0, 0)),
                  pl.BlockSpec((k, m), lambda i: (0, 0)),
                  pl.BlockSpec((1, m), lambda i: (0, 0))],
        out_specs=pl.BlockSpec((n, m), lambda i: (0, 0)),
        out_shape=jax.ShapeDtypeStruct((n, m), jnp.float32),
    )(p, w, b)


def kernel(x, edge_index, W1, b1, W2, b2):
    n_nodes, _ = x.shape
    n_edges = edge_index.shape[1]
    d_hid = W1.shape[1]

    align = 8 * NW
    n_pad = (n_nodes + align - 1) // align * align

    epw = n_edges // NW
    nch = epw // CHUNK
    edges = edge_index.astype(jnp.int32).reshape(2, NW, nch, CHUNK)

    seg = _seg_sum_kernel(n_pad, d_hid, n_edges)
    relu_k = _relu_combine_sc_kernel(n_pad, d_hid)

    h1 = _mm(x, W1)
    p1 = seg(h1, edges)
    h = relu_k(p1, b1)
    p2 = seg(h, edges)
    return _combine_mm_bias(p2, W2, b2.reshape(1, -1), n=n_nodes)


# speedup vs baseline: 33.1745x; 1.0637x over previous
import functools

import jax
import jax.numpy as jnp
from jax import lax
from jax.experimental import pallas as pl
from jax.experimental.pallas import tpu as pltpu
from jax.experimental.pallas import tpu_sc as plsc

NC = 2
NS = 16
NW = NC * NS
CHUNK = 125
NBUF = 10


@functools.cache
def _seg_sum_kernel(n_nodes, d, n_edges):
    epw = n_edges // NW
    nch = epw // CHUNK
    rpt = n_nodes // NS
    mesh = plsc.VectorSubcoreMesh(core_axis_name="c", subcore_axis_name="s")

    assert nch % NBUF == 0 and nch // NBUF >= 2

    @functools.partial(
        pl.kernel,
        out_type=jax.ShapeDtypeStruct((NC, n_nodes, d), jnp.float32),
        mesh=mesh,
        compiler_params=pltpu.CompilerParams(use_tc_tiling_on_sc=False),
        scratch_types=[
            pltpu.VMEM((nch, CHUNK), jnp.int32),
            pltpu.VMEM((nch, CHUNK), jnp.int32),
            pltpu.VMEM((NBUF, CHUNK, d), jnp.float32),
            pltpu.VMEM((rpt, d), jnp.float32),
            pltpu.VMEM_SHARED((n_nodes, d), jnp.float32),
        ] + [pltpu.SemaphoreType.DMA] * (2 * NBUF),
    )
    def seg(table_hbm, edge_hbm, out_hbm,
            src_v, dst_v, rows_v, zero_v, acc_sh, *sems):
        cid = lax.axis_index("c")
        sid = lax.axis_index("s")
        wid = sid * NC + cid

        pltpu.sync_copy(edge_hbm.at[0, wid], src_v)
        pltpu.sync_copy(edge_hbm.at[1, wid], dst_v)

        gsems = sems[:NBUF]
        ssems = sems[NBUF:]

        def gather(j, b):
            pltpu.async_copy(table_hbm.at[src_v.at[j]], rows_v.at[b], gsems[b])

        def gwait(b):
            pltpu.make_async_copy(table_hbm.at[src_v.at[0]],
                                  rows_v.at[b], gsems[b]).wait()

        def scatter(j, b):
            pltpu.async_copy(rows_v.at[b], acc_sh.at[dst_v.at[j]], ssems[b],
                             add=True)

        def swait(b):
            pltpu.make_async_copy(rows_v.at[b], acc_sh.at[dst_v.at[0]],
                                  ssems[b]).wait()

        for b in range(NBUF):
            gather(b, b)

        zrow = jnp.zeros((d,), jnp.float32)

        def zbody(i, carry):
            for u in range(8):
                zero_v[i * 8 + u] = zrow
            return carry

        lax.fori_loop(0, rpt // 8, zbody, 0)
        row0 = sid * rpt
        pltpu.sync_copy(zero_v, acc_sh.at[pl.ds(row0, rpt)])
        plsc.subcore_barrier()

        def body(g, carry):
            j0 = g * NBUF
            for b in range(NBUF):
                gwait(b)
                scatter(j0 + b, b)
            for b in range(NBUF):
                swait(b)
                gather(j0 + b + NBUF, b)
            return carry

        lax.fori_loop(0, nch // NBUF - 1, body, 0)
        j0 = nch - NBUF
        for b in range(NBUF):
            gwait(b)
            scatter(j0 + b, b)
        for b in range(NBUF):
            swait(b)
        plsc.subcore_barrier()

        pltpu.sync_copy(acc_sh.at[pl.ds(row0, rpt)],
                        out_hbm.at[cid, pl.ds(row0, rpt)])

    return seg


@functools.cache
def _relu_combine_sc_kernel(n_nodes, d):
    assert d == 16 and n_nodes % (8 * NW) == 0
    rpt = n_nodes // NW
    mesh = plsc.VectorSubcoreMesh(core_axis_name="c", subcore_axis_name="s")

    @functools.partial(
        pl.kernel,
        out_type=jax.ShapeDtypeStruct((n_nodes, d), jnp.float32),
        mesh=mesh,
        compiler_params=pltpu.CompilerParams(use_tc_tiling_on_sc=False),
        scratch_types=[
            pltpu.VMEM((rpt, d), jnp.float32),
            pltpu.VMEM((rpt, d), jnp.float32),
            pltpu.VMEM((d,), jnp.float32),
        ],
    )
    def relu_k(p_hbm, b_hbm, out_hbm, a_v, c_v, bias_v):
        cid = lax.axis_index("c")
        sid = lax.axis_index("s")
        wid = sid * NC + cid
        r0 = wid * rpt
        pltpu.sync_copy(p_hbm.at[0, pl.ds(r0, rpt)], a_v)
        pltpu.sync_copy(p_hbm.at[1, pl.ds(r0, rpt)], c_v)
        pltpu.sync_copy(b_hbm, bias_v)
        bias = bias_v[...]

        UNROLL = 8

        def body(i, carry):
            for u in range(UNROLL):
                r = i * UNROLL + u
                a_v[r] = jnp.maximum(a_v[r] + c_v[r] + bias, 0.0)
            return carry

        lax.fori_loop(0, rpt // UNROLL, body, 0)
        pltpu.sync_copy(a_v, out_hbm.at[pl.ds(r0, rpt)])

    return relu_k


def _mm(x, w):
    n, k = x.shape
    m = w.shape[1]

    def body(x_ref, w_ref, o_ref):
        o_ref[...] = jnp.dot(x_ref[...], w_ref[...],
                             preferred_element_type=jnp.float32)

    return pl.pallas_call(
        body,
        out_shape=jax.ShapeDtypeStruct((n, m), jnp.float32),
    )(x, w)


def _combine_mm_bias(p, w, b, n):
    npad, k = p.shape[1], p.shape[2]
    m = w.shape[1]

    def body(p_ref, w_ref, b_ref, o_ref):
        s = p_ref[0] + p_ref[1]
        o_ref[...] = jnp.dot(s, w_ref[...],
                             preferred_element_type=jnp.float32) + b_ref[...]

    return pl.pallas_call(
        body,
        grid=(1,),
        in_specs=[pl.BlockSpec((2, n, k), lambda i: (0, 0, 0)),
                  pl.BlockSpec((k, m), lambda i: (0, 0)),
                  pl.BlockSpec((1, m), lambda i: (0, 0))],
        out_specs=pl.BlockSpec((n, m), lambda i: (0, 0)),
        out_shape=jax.ShapeDtypeStruct((n, m), jnp.float32),
    )(p, w, b)


def kernel(x, edge_index, W1, b1, W2, b2):
    n_nodes, _ = x.shape
    n_edges = edge_index.shape[1]
    d_hid = W1.shape[1]

    align = 8 * NW
    n_pad = (n_nodes + align - 1) // align * align

    epw = n_edges // NW
    nch = epw // CHUNK
    edges = edge_index.astype(jnp.int32).reshape(2, NW, nch, CHUNK)

    seg = _seg_sum_kernel(n_pad, d_hid, n_edges)
    relu_k = _relu_combine_sc_kernel(n_pad, d_hid)

    h1 = _mm(x, W1)
    p1 = seg(h1, edges)
    h = relu_k(p1, b1)
    p2 = seg(h, edges)
    return _combine_mm_bias(p2, W2, b2.reshape(1, -1), n=n_nodes)
